# baseline (speedup 1.0000x reference)
import functools

import jax
import jax.numpy as jnp
from jax import lax
from jax.experimental import pallas as pl
from jax.experimental.pallas import tpu as pltpu
from jax.experimental.pallas import tpu_sc as plsc

_N = 8192
_C = 256
_BLOCK_ROWS = 512
_GRID = _N // _BLOCK_ROWS
_NC = 2
_NS = 16
_NW = _NC * _NS
_CHUNK = _N // _NW
_L = 16


_SCAT_UNROLL = 16


def _sc_weights_body(t_hbm, z_hbm, c_hbm, t_v, hist_v, cntf_v, cout_v):
    wid = lax.axis_index("s") * _NC + lax.axis_index("c")
    pltpu.sync_copy(z_hbm, hist_v)
    pltpu.sync_copy(t_hbm, t_v)

    lane_base = lax.iota(jnp.int32, _L) * _C
    ones = jnp.ones((_L,), jnp.int32)

    def scat_body(j, carry):
        for u in range(_SCAT_UNROLL):
            t16 = t_v[pl.ds((j * _SCAT_UNROLL + u) * _L, _L)]
            plsc.addupdate_scatter(hist_v, [lane_base + t16], ones)
        return carry

    lax.fori_loop(0, _N // (_L * _SCAT_UNROLL), scat_body, 0)

    for j in range(_C // _L):
        acc = hist_v[pl.ds(j * _L, _L)]
        for l in range(1, _L):
            acc = acc + hist_v[pl.ds(l * _C + j * _L, _L)]
        cntf_v[pl.ds(j * _L, _L)] = acc.astype(jnp.float32)

    base = wid * _CHUNK
    for v in range(_CHUNK // _L):
        t16 = t_v[pl.ds(base + v * _L, _L)]
        cnt = plsc.load_gather(cntf_v, [t16])
        cout_v[pl.ds(v * _L, _L)] = t16.astype(jnp.float32) / cnt
    pltpu.sync_copy(cout_v, c_hbm.at[pl.ds(base, _CHUNK)])


_sc_weights_call = functools.partial(
    pl.kernel,
    mesh=plsc.VectorSubcoreMesh(core_axis_name="c", subcore_axis_name="s"),
    out_type=jax.ShapeDtypeStruct((_N,), jnp.float32),
    compiler_params=pltpu.CompilerParams(
        needs_layout_passes=False, skip_device_barrier=True),
    scratch_types=[
        pltpu.VMEM((_N,), jnp.int32),
        pltpu.VMEM((_L * _C,), jnp.int32),
        pltpu.VMEM((_C,), jnp.float32),
        pltpu.VMEM((_CHUNK,), jnp.float32),
    ],
)(_sc_weights_body)


def _sc_weights(target):
    zeros = jnp.zeros((_L * _C,), jnp.int32)
    return _sc_weights_call(target, zeros)


def _stream_kernel(x_ref, lse_ref, colsum_ref):
    i = pl.program_id(0)

    @pl.when(i == 0)
    def _init():
        lse_ref[...] = jnp.zeros_like(lse_ref)
        colsum_ref[...] = jnp.zeros_like(colsum_ref)

    x = x_ref[...]
    m = jnp.max(x, axis=1, keepdims=True)
    lse = jnp.log(jnp.sum(jnp.exp(x - m), axis=1, keepdims=True)) + m
    lse_ref[...] += jnp.full((1, 1), 1.0, jnp.float32) * jnp.sum(lse)
    colsum_ref[...] += jnp.sum(x, axis=0, keepdims=True)


def _combine_kernel(c_ref, colsum_ref, lse_ref, out_ref):
    c = c_ref[...]
    s = jnp.sum(c)
    wdot = jnp.sum(c * colsum_ref[...])
    out_ref[...] = (s * lse_ref[...] - wdot) * (1.0 / _N)


def _stream_call(x):
    rows = x.shape[0]
    return pl.pallas_call(
        _stream_kernel,
        grid=(rows // _BLOCK_ROWS,),
        in_specs=[
            pl.BlockSpec((_BLOCK_ROWS, _N), lambda i: (i, 0)),
        ],
        out_specs=[
            pl.BlockSpec((1, 1), lambda i: (0, 0)),
            pl.BlockSpec((1, _N), lambda i: (0, 0)),
        ],
        out_shape=[
            jax.ShapeDtypeStruct((1, 1), jnp.float32),
            jax.ShapeDtypeStruct((1, _N), jnp.float32),
        ],
    )(x)


def kernel(input, target):
    half = _N // 2
    lse_a, colsum_a = _stream_call(input[:half])
    c = _sc_weights(target)
    lse_b, colsum_b = _stream_call(input[half:])
    out = pl.pallas_call(
        _combine_kernel,
        out_shape=jax.ShapeDtypeStruct((1, 1), jnp.float32),
    )(c.reshape(1, _N), colsum_a + colsum_b, lse_a + lse_b)
    return out.reshape(())


# speedup vs baseline: 2.4776x; 2.4776x over previous
import functools

import jax
import jax.numpy as jnp
from jax import lax
from jax.experimental import pallas as pl
from jax.experimental.pallas import tpu as pltpu
from jax.experimental.pallas import tpu_sc as plsc

_N = 8192
_C = 256
_BLOCK_ROWS = 512
_GRID = _N // _BLOCK_ROWS
_NC = 2
_NS = 16
_NW = _NC * _NS
_CHUNK = _N // _NW
_L = 16


_SCAT_UNROLL = 16


def _sc_weights_body(t_hbm, z_hbm, c_hbm, t_v, hist_v, cntf_v, cout_v):
    wid = lax.axis_index("s") * _NC + lax.axis_index("c")
    pltpu.sync_copy(z_hbm, hist_v)
    pltpu.sync_copy(t_hbm, t_v)

    lane_base = lax.iota(jnp.int32, _L) * _C
    ones = jnp.ones((_L,), jnp.int32)

    def scat_body(j, carry):
        for u in range(_SCAT_UNROLL):
            t16 = t_v[pl.ds((j * _SCAT_UNROLL + u) * _L, _L)]
            plsc.addupdate_scatter(hist_v, [lane_base + t16], ones)
        return carry

    lax.fori_loop(0, _N // (_L * _SCAT_UNROLL), scat_body, 0)

    for j in range(_C // _L):
        acc = hist_v[pl.ds(j * _L, _L)]
        for l in range(1, _L):
            acc = acc + hist_v[pl.ds(l * _C + j * _L, _L)]
        cntf_v[pl.ds(j * _L, _L)] = acc.astype(jnp.float32)

    base = wid * _CHUNK
    for v in range(_CHUNK // _L):
        t16 = t_v[pl.ds(base + v * _L, _L)]
        cnt = plsc.load_gather(cntf_v, [t16])
        cout_v[pl.ds(v * _L, _L)] = t16.astype(jnp.float32) / cnt
    pltpu.sync_copy(cout_v, c_hbm.at[pl.ds(base, _CHUNK)])


_sc_weights_call = functools.partial(
    pl.kernel,
    mesh=plsc.VectorSubcoreMesh(core_axis_name="c", subcore_axis_name="s"),
    out_type=jax.ShapeDtypeStruct((_N,), jnp.float32),
    compiler_params=pltpu.CompilerParams(
        needs_layout_passes=False, skip_device_barrier=True),
    scratch_types=[
        pltpu.VMEM((_N,), jnp.int32),
        pltpu.VMEM((_L * _C,), jnp.int32),
        pltpu.VMEM((_C,), jnp.float32),
        pltpu.VMEM((_CHUNK,), jnp.float32),
    ],
)(_sc_weights_body)


def _sc_weights(target):
    zeros = jnp.zeros((_L * _C,), jnp.int32)
    return _sc_weights_call(target, zeros)


def _stream_kernel(x_ref, lse_ref, colsum_ref):
    i = pl.program_id(0)

    @pl.when(i == 0)
    def _init():
        lse_ref[...] = jnp.zeros_like(lse_ref)
        colsum_ref[...] = jnp.zeros_like(colsum_ref)

    x = x_ref[...]
    m = jnp.max(x, axis=1, keepdims=True)
    lse = jnp.log(jnp.sum(jnp.exp(x - m), axis=1, keepdims=True)) + m
    lse_ref[...] += jnp.full((1, 1), 1.0, jnp.float32) * jnp.sum(lse)
    colsum_ref[...] += jnp.sum(x, axis=0, keepdims=True)


def _combine_kernel(c_ref, colsum_ref, lse_ref, out_ref):
    c = c_ref[...]
    s = jnp.sum(c)
    wdot = jnp.sum(c * colsum_ref[...])
    out_ref[...] = (s * lse_ref[...] - wdot) * (1.0 / _N)


def _stream_call(x, block_offset):
    return pl.pallas_call(
        _stream_kernel,
        grid=(_GRID // 2,),
        in_specs=[
            pl.BlockSpec((_BLOCK_ROWS, _N), lambda i: (i + block_offset, 0)),
        ],
        out_specs=[
            pl.BlockSpec((1, 1), lambda i: (0, 0)),
            pl.BlockSpec((1, _N), lambda i: (0, 0)),
        ],
        out_shape=[
            jax.ShapeDtypeStruct((1, 1), jnp.float32),
            jax.ShapeDtypeStruct((1, _N), jnp.float32),
        ],
    )(x)


def kernel(input, target):
    lse_a, colsum_a = _stream_call(input, 0)
    c = _sc_weights(target)
    lse_b, colsum_b = _stream_call(input, _GRID // 2)
    out = pl.pallas_call(
        _combine_kernel,
        out_shape=jax.ShapeDtypeStruct((1, 1), jnp.float32),
    )(c.reshape(1, _N), colsum_a + colsum_b, lse_a + lse_b)
    return out.reshape(())


# speedup vs baseline: 2.7061x; 1.0922x over previous
import functools

import jax
import jax.numpy as jnp
from jax import lax
from jax.experimental import pallas as pl
from jax.experimental.pallas import tpu as pltpu
from jax.experimental.pallas import tpu_sc as plsc

_N = 8192
_C = 256
_BLOCK_ROWS = 512
_GRID = _N // _BLOCK_ROWS
_NC = 2
_NS = 16
_NW = _NC * _NS
_CHUNK = _N // _NW
_L = 16


def _sc_weights_body(t_hbm, c_hbm, t_v, hist_v, cntf_v, cout_v):
    wid = lax.axis_index("s") * _NC + lax.axis_index("c")
    pltpu.sync_copy(t_hbm, t_v)

    def zero_body(j, carry):
        hist_v[pl.ds(j * _L, _L)] = jnp.zeros((_L,), jnp.int32)
        return carry

    lax.fori_loop(0, (_L * _C) // _L, zero_body, 0)

    lane_base = lax.iota(jnp.int32, _L) * _C
    ones = jnp.ones((_L,), jnp.int32)

    def scat_body(j, carry):
        t16 = t_v[pl.ds(j * _L, _L)]
        plsc.addupdate_scatter(hist_v, [lane_base + t16], ones)
        return carry

    lax.fori_loop(0, _N // _L, scat_body, 0)

    def red_body(j, carry):
        acc = hist_v[pl.ds(j * _L, _L)]
        for l in range(1, _L):
            acc = acc + hist_v[pl.ds(l * _C + j * _L, _L)]
        cntf_v[pl.ds(j * _L, _L)] = acc.astype(jnp.float32)
        return carry

    lax.fori_loop(0, _C // _L, red_body, 0)

    base = wid * _CHUNK

    def w_body(v, carry):
        t16 = t_v[pl.ds(base + v * _L, _L)]
        cnt = plsc.load_gather(cntf_v, [t16])
        cout_v[pl.ds(v * _L, _L)] = t16.astype(jnp.float32) / cnt
        return carry

    lax.fori_loop(0, _CHUNK // _L, w_body, 0)
    pltpu.sync_copy(cout_v, c_hbm.at[pl.ds(base, _CHUNK)])


_sc_weights = functools.partial(
    pl.kernel,
    mesh=plsc.VectorSubcoreMesh(core_axis_name="c", subcore_axis_name="s"),
    out_type=jax.ShapeDtypeStruct((_N,), jnp.float32),
    compiler_params=pltpu.CompilerParams(
        needs_layout_passes=False, skip_device_barrier=True),
    scratch_types=[
        pltpu.VMEM((_N,), jnp.int32),
        pltpu.VMEM((_L * _C,), jnp.int32),
        pltpu.VMEM((_C,), jnp.float32),
        pltpu.VMEM((_CHUNK,), jnp.float32),
    ],
)(_sc_weights_body)


def _stream_kernel(x_ref, lse_ref, colsum_ref):
    i = pl.program_id(0)

    @pl.when(i == 0)
    def _init():
        lse_ref[...] = jnp.zeros_like(lse_ref)
        colsum_ref[...] = jnp.zeros_like(colsum_ref)

    x = x_ref[...]
    m = jnp.max(x, axis=1, keepdims=True)
    lse = jnp.log(jnp.sum(jnp.exp(x - m), axis=1, keepdims=True)) + m
    lse_ref[...] += jnp.full((1, 1), 1.0, jnp.float32) * jnp.sum(lse)
    colsum_ref[...] += jnp.sum(x, axis=0, keepdims=True)


def _combine_kernel(c_ref, colsum_ref, lse_ref, out_ref):
    c = c_ref[...]
    s = jnp.sum(c)
    wdot = jnp.sum(c * colsum_ref[...])
    out_ref[...] = (s * lse_ref[...] - wdot) * (1.0 / _N)


def kernel(input, target):
    c = _sc_weights(target)
    lse_sum, colsum = pl.pallas_call(
        _stream_kernel,
        grid=(_GRID,),
        in_specs=[
            pl.BlockSpec((_BLOCK_ROWS, _N), lambda i: (i, 0)),
        ],
        out_specs=[
            pl.BlockSpec((1, 1), lambda i: (0, 0)),
            pl.BlockSpec((1, _N), lambda i: (0, 0)),
        ],
        out_shape=[
            jax.ShapeDtypeStruct((1, 1), jnp.float32),
            jax.ShapeDtypeStruct((1, _N), jnp.float32),
        ],
        compiler_params=pltpu.CompilerParams(skip_device_barrier=True),
    )(input)
    out = pl.pallas_call(
        _combine_kernel,
        out_shape=jax.ShapeDtypeStruct((1, 1), jnp.float32),
        compiler_params=pltpu.CompilerParams(skip_device_barrier=True),
    )(c.reshape(1, _N), colsum, lse_sum)
    return out.reshape(())


# speedup vs baseline: 2.7978x; 1.0339x over previous
import functools

import jax
import jax.numpy as jnp
from jax import lax
from jax.experimental import pallas as pl
from jax.experimental.pallas import tpu as pltpu
from jax.experimental.pallas import tpu_sc as plsc

_N = 8192
_C = 256
_BLOCK_ROWS = 512
_GRID = _N // _BLOCK_ROWS
_NC = 1
_NS = 16
_NW = _NC * _NS
_CHUNK = _N // _NW
_L = 16


def _sc_weights_body(t_hbm, c_hbm, t_v, hist_v, cntf_v, cout_v):
    wid = lax.axis_index("s") * _NC + lax.axis_index("c")
    pltpu.sync_copy(t_hbm, t_v)

    def zero_body(j, carry):
        hist_v[pl.ds(j * _L, _L)] = jnp.zeros((_L,), jnp.int32)
        return carry

    lax.fori_loop(0, (_L * _C) // _L, zero_body, 0)

    lane_base = lax.iota(jnp.int32, _L) * _C
    ones = jnp.ones((_L,), jnp.int32)

    def scat_body(j, carry):
        t16 = t_v[pl.ds(j * _L, _L)]
        plsc.addupdate_scatter(hist_v, [lane_base + t16], ones)
        return carry

    lax.fori_loop(0, _N // _L, scat_body, 0)

    def red_body(j, carry):
        acc = hist_v[pl.ds(j * _L, _L)]
        for l in range(1, _L):
            acc = acc + hist_v[pl.ds(l * _C + j * _L, _L)]
        cntf_v[pl.ds(j * _L, _L)] = acc.astype(jnp.float32)
        return carry

    lax.fori_loop(0, _C // _L, red_body, 0)

    base = wid * _CHUNK

    def w_body(v, carry):
        t16 = t_v[pl.ds(base + v * _L, _L)]
        cnt = plsc.load_gather(cntf_v, [t16])
        cout_v[pl.ds(v * _L, _L)] = t16.astype(jnp.float32) / cnt
        return carry

    lax.fori_loop(0, _CHUNK // _L, w_body, 0)
    pltpu.sync_copy(cout_v, c_hbm.at[pl.ds(base, _CHUNK)])


_sc_weights = functools.partial(
    pl.kernel,
    mesh=plsc.VectorSubcoreMesh(core_axis_name="c", subcore_axis_name="s", num_cores=1),
    out_type=jax.ShapeDtypeStruct((_N,), jnp.float32),
    compiler_params=pltpu.CompilerParams(
        needs_layout_passes=False, skip_device_barrier=True),
    scratch_types=[
        pltpu.VMEM((_N,), jnp.int32),
        pltpu.VMEM((_L * _C,), jnp.int32),
        pltpu.VMEM((_C,), jnp.float32),
        pltpu.VMEM((_CHUNK,), jnp.float32),
    ],
)(_sc_weights_body)


def _stream_kernel(x_ref, lse_ref, colsum_ref):
    i = pl.program_id(0)

    @pl.when(i == 0)
    def _init():
        lse_ref[...] = jnp.zeros_like(lse_ref)
        colsum_ref[...] = jnp.zeros_like(colsum_ref)

    x = x_ref[...]
    m = jnp.max(x, axis=1, keepdims=True)
    lse = jnp.log(jnp.sum(jnp.exp(x - m), axis=1, keepdims=True)) + m
    lse_ref[...] += jnp.full((1, 1), 1.0, jnp.float32) * jnp.sum(lse)
    colsum_ref[...] += jnp.sum(x, axis=0, keepdims=True)


def _combine_kernel(c_ref, colsum_ref, lse_ref, out_ref):
    c = c_ref[...]
    s = jnp.sum(c)
    wdot = jnp.sum(c * colsum_ref[...])
    out_ref[...] = (s * lse_ref[...] - wdot) * (1.0 / _N)


def kernel(input, target):
    c = _sc_weights(target)
    lse_sum, colsum = pl.pallas_call(
        _stream_kernel,
        grid=(_GRID,),
        in_specs=[
            pl.BlockSpec((_BLOCK_ROWS, _N), lambda i: (i, 0)),
        ],
        out_specs=[
            pl.BlockSpec((1, 1), lambda i: (0, 0)),
            pl.BlockSpec((1, _N), lambda i: (0, 0)),
        ],
        out_shape=[
            jax.ShapeDtypeStruct((1, 1), jnp.float32),
            jax.ShapeDtypeStruct((1, _N), jnp.float32),
        ],
        compiler_params=pltpu.CompilerParams(skip_device_barrier=True),
    )(input)
    out = pl.pallas_call(
        _combine_kernel,
        out_shape=jax.ShapeDtypeStruct((1, 1), jnp.float32),
        compiler_params=pltpu.CompilerParams(skip_device_barrier=True),
    )(c.reshape(1, _N), colsum, lse_sum)
    return out.reshape(())
